# baseline (speedup 1.0000x reference)
import functools

import jax
import jax.numpy as jnp
from jax import lax
from jax.experimental import pallas as pl
from jax.experimental.pallas import tpu as pltpu
from jax.experimental.pallas import tpu_sc as plsc

NUM_TYPES = 1000000
CHANNELS = 64
B = 16384
L = 200

NC = 2
NS = 16
NW = NC * NS

IDX_COLS = 128
N_FLAT = B * L
N_ROWS = N_FLAT // IDX_COLS
ROWS_PER_W = N_ROWS // NW
K = 8
NCH = ROWS_PER_W // K


def _sc_gather(types2d, table):
    mesh = plsc.VectorSubcoreMesh(core_axis_name="c", subcore_axis_name="s")

    @functools.partial(
        pl.kernel,
        mesh=mesh,
        out_type=jax.ShapeDtypeStruct((N_ROWS, IDX_COLS, CHANNELS), jnp.float32),
        scratch_types=[
            pltpu.VMEM((K, IDX_COLS), jnp.int32),
            pltpu.VMEM((K, IDX_COLS, CHANNELS), jnp.float32),
            pltpu.SemaphoreType.DMA,
        ],
        compiler_params=pltpu.CompilerParams(use_tc_tiling_on_sc=False),
    )
    def run(idx_hbm, table_hbm, out_hbm, idx_v, rows_v, sem):
        wid = lax.axis_index("s") * NC + lax.axis_index("c")
        w_base = wid * ROWS_PER_W

        def chunk(c, carry):
            base = w_base + c * K
            pltpu.sync_copy(idx_hbm.at[pl.ds(base, K)], idx_v)
            handles = [
                pltpu.async_copy(table_hbm.at[idx_v.at[j]], rows_v.at[j], sem)
                for j in range(K)
            ]
            for h in handles:
                h.wait()
            pltpu.sync_copy(rows_v, out_hbm.at[pl.ds(base, K)])
            return carry

        lax.fori_loop(0, NCH, chunk, 0)

    return run(types2d, table)


def kernel(types, table):
    types2d = types.reshape(N_ROWS, IDX_COLS)
    out = _sc_gather(types2d, table)
    return out.reshape(B, L, CHANNELS)


# speedup vs baseline: 1.0229x; 1.0229x over previous
import functools

import jax
import jax.numpy as jnp
from jax import lax
from jax.experimental import pallas as pl
from jax.experimental.pallas import tpu as pltpu
from jax.experimental.pallas import tpu_sc as plsc

NUM_TYPES = 1000000
CHANNELS = 64
B = 16384
L = 200

NC = 2
NS = 16
NW = NC * NS

GW = 128
N_FLAT = B * L
K = 5
KI = K * GW
PER_W = N_FLAT // NW
NCH = PER_W // KI


def _sc_gather(types_flat, table):
    mesh = plsc.VectorSubcoreMesh(core_axis_name="c", subcore_axis_name="s")

    @functools.partial(
        pl.kernel,
        mesh=mesh,
        out_type=jax.ShapeDtypeStruct((N_FLAT, CHANNELS), jnp.float32),
        scratch_types=[
            pltpu.VMEM((2, KI), jnp.int32),
            pltpu.VMEM((2, KI, CHANNELS), jnp.float32),
            pltpu.SemaphoreType.DMA,
            pltpu.SemaphoreType.DMA,
            pltpu.SemaphoreType.DMA,
        ],
        compiler_params=pltpu.CompilerParams(use_tc_tiling_on_sc=False),
    )
    def run(idx_hbm, table_hbm, out_hbm, idx_v, rows_v, sem_g, sem_o0, sem_o1):
        wid = lax.axis_index("s") * NC + lax.axis_index("c")
        w_base = wid * PER_W
        sems_o = (sem_o0, sem_o1)

        def fire_chunk(c, buf):
            base = w_base + c * KI
            pltpu.sync_copy(idx_hbm.at[pl.ds(base, KI)], idx_v.at[buf])
            for j in range(K):
                pltpu.async_copy(
                    table_hbm.at[idx_v.at[buf, pl.ds(j * GW, GW)]],
                    rows_v.at[buf, pl.ds(j * GW, GW)],
                    sem_g,
                )

        def drain_gathers(c, buf):
            for j in range(K):
                pltpu.make_async_copy(
                    table_hbm.at[idx_v.at[buf, pl.ds(j * GW, GW)]],
                    rows_v.at[buf, pl.ds(j * GW, GW)],
                    sem_g,
                ).wait()

        def out_copy(c, buf):
            base = w_base + c * KI
            pltpu.async_copy(
                rows_v.at[buf], out_hbm.at[pl.ds(base, KI)], sems_o[buf]
            )

        def wait_out(c, buf):
            base = w_base + c * KI
            pltpu.make_async_copy(
                rows_v.at[buf], out_hbm.at[pl.ds(base, KI)], sems_o[buf]
            ).wait()

        fire_chunk(0, 0)

        def step(c, carry):
            def body_for(buf):
                nbuf = 1 - buf
                drain_gathers(c, buf)
                out_copy(c, buf)

                @pl.when(c + 1 < NCH)
                def _():
                    @pl.when(c >= 1)
                    def _():
                        wait_out(c - 1, nbuf)

                    fire_chunk(c + 1, nbuf)

            lax.cond(c % 2 == 0, lambda: body_for(0), lambda: body_for(1))
            return carry

        lax.fori_loop(0, NCH, step, 0)
        wait_out(NCH - 2, (NCH - 2) % 2)
        wait_out(NCH - 1, (NCH - 1) % 2)

    return run(types_flat, table)


def kernel(types, table):
    out = _sc_gather(types.reshape(N_FLAT), table)
    return out.reshape(B, L, CHANNELS)


# speedup vs baseline: 1.6804x; 1.6428x over previous
import functools

import jax
import jax.numpy as jnp
from jax import lax
from jax.experimental import pallas as pl
from jax.experimental.pallas import tpu as pltpu
from jax.experimental.pallas import tpu_sc as plsc

NUM_TYPES = 1000000
CHANNELS = 64
B = 16384
L = 200

NC = 2
NS = 16
NW = NC * NS

GW = 128
N_FLAT = B * L
K = 5
KI = K * GW
PER_W = N_FLAT // NW
NCH = PER_W // KI


def _sc_gather(types_flat, table):
    mesh = plsc.VectorSubcoreMesh(core_axis_name="c", subcore_axis_name="s")

    @functools.partial(
        pl.kernel,
        mesh=mesh,
        out_type=jax.ShapeDtypeStruct((N_FLAT, 2 * CHANNELS), jnp.float32),
        scratch_types=[
            pltpu.VMEM((2, KI), jnp.int32),
            pltpu.VMEM((2, KI, CHANNELS), jnp.float32),
            pltpu.SemaphoreType.DMA,
            pltpu.SemaphoreType.DMA,
            pltpu.SemaphoreType.DMA,
        ],
        compiler_params=pltpu.CompilerParams(use_tc_tiling_on_sc=False),
    )
    def run(idx_hbm, table_hbm, out_hbm, idx_v, rows_v, sem_g, sem_o0, sem_o1):
        wid = lax.axis_index("s") * NC + lax.axis_index("c")
        w_base = wid * PER_W
        sems_o = (sem_o0, sem_o1)

        def fire_chunk(c, buf):
            base = w_base + c * KI
            pltpu.sync_copy(idx_hbm.at[pl.ds(base, KI)], idx_v.at[buf])
            for j in range(K):
                pltpu.async_copy(
                    table_hbm.at[idx_v.at[buf, pl.ds(j * GW, GW)]],
                    rows_v.at[buf, pl.ds(j * GW, GW)],
                    sem_g,
                )

        def drain_gathers(c, buf):
            for j in range(K):
                pltpu.make_async_copy(
                    table_hbm.at[idx_v.at[buf, pl.ds(j * GW, GW)]],
                    rows_v.at[buf, pl.ds(j * GW, GW)],
                    sem_g,
                ).wait()

        def out_copy(c, buf):
            base = w_base + c * KI
            pltpu.async_copy(
                rows_v.at[buf],
                out_hbm.at[pl.ds(base, KI), pl.ds(0, CHANNELS)],
                sems_o[buf],
            )

        def wait_out(c, buf):
            base = w_base + c * KI
            pltpu.make_async_copy(
                rows_v.at[buf],
                out_hbm.at[pl.ds(base, KI), pl.ds(0, CHANNELS)],
                sems_o[buf],
            ).wait()

        fire_chunk(0, 0)

        def step(c, carry):
            def body_for(buf):
                nbuf = 1 - buf
                drain_gathers(c, buf)
                out_copy(c, buf)

                @pl.when(c + 1 < NCH)
                def _():
                    @pl.when(c >= 1)
                    def _():
                        wait_out(c - 1, nbuf)

                    fire_chunk(c + 1, nbuf)

            lax.cond(c % 2 == 0, lambda: body_for(0), lambda: body_for(1))
            return carry

        lax.fori_loop(0, NCH, step, 0)
        wait_out(NCH - 2, (NCH - 2) % 2)
        wait_out(NCH - 1, (NCH - 1) % 2)

    return run(types_flat, table)


def kernel(types, table):
    out = _sc_gather(types.reshape(N_FLAT), table)
    return out[:, :CHANNELS].reshape(B, L, CHANNELS)
